# Initial kernel scaffold; baseline (speedup 1.0000x reference)
import jax
import jax.numpy as jnp
from jax.experimental import pallas as pl


def kernel(global_feature, map_feature, factory_feature, unit_feature, location_feature, va_factory_act, va_move, va_transfer, va_pickup, va_dig, va_self_destruct, va_recharge, va_do_nothing, W_factory, b_factory, W_critic, b_critic, W_dir, b_dir, W_act, b_act):
    raise NotImplementedError("write your pallas kernel here")


# speedup vs baseline: 10.7296x; 10.7296x over previous
import functools

import jax
import jax.numpy as jnp
from jax import lax
from jax.experimental import pallas as pl
from jax.experimental.pallas import tpu as pltpu
from jax.experimental.pallas import tpu_sc as plsc

_B = 256
_H = 48
_W = 48
_P = _H * _W
_NIDS = 1000
_NPAD = 1008
_NEG = -1e9


def _mm(a, b):
    return jax.lax.dot_general(a, b, (((1,), (0,)), ((), ())),
                               preferred_element_type=jnp.float32)


def _dense_body(mapf_ref, unitf_ref, factf_ref, loc1_ref,
                vfact_ref, vmove_ref, vtrans_ref, vpick_ref, vdig_ref,
                vsd_ref, vrech_ref, vdn_ref,
                w_au_ref, b_au_ref, w_fact_ref, b_fact_ref,
                ulogp_ref, uent_ref, critic_ref, ids_ref,
                factmap_ref, uactmap_ref):
    f32 = jnp.float32
    mapf = mapf_ref[0]
    unitf = unitf_ref[0]
    value = jnp.concatenate([mapf, unitf], axis=0)

    mask80 = jnp.concatenate(
        [vmove_ref[0].astype(f32), vtrans_ref[0].astype(f32),
         vpick_ref[0].astype(f32), vdig_ref[0].astype(f32),
         vsd_ref[0].astype(f32), vrech_ref[0].astype(f32),
         vdn_ref[0].astype(f32), vfact_ref[0].astype(f32)], axis=0)
    kk = lax.broadcasted_iota(jnp.int32, (16, 80), 0)
    cc = lax.broadcasted_iota(jnp.int32, (16, 80), 1)
    gid = ((cc >= 10).astype(jnp.int32) + (cc >= 60) + (cc >= 70)
           + (cc >= 72) + (cc >= 73) + (cc >= 75) + (cc >= 76))
    sel = (((kk == gid) & (kk < 7)) | ((kk == 7) & (cc < 76))
           | ((kk == 8) & (cc >= 76))).astype(f32)
    sums = _mm(sel, mask80)
    ua = sums[0:7] != 0.0
    umask = sums[7:8] != 0.0
    fmask = sums[8:9] != 0.0
    fva = mask80[76:80] != 0.0

    logits8 = _mm(w_au_ref[...], value) + b_au_ref[...]
    logits = logits8[0:7]
    critic = logits8[7:8]
    neg = jnp.where(ua, logits, _NEG)
    m = jnp.max(neg, axis=0, keepdims=True)
    e = jnp.exp(neg - m)
    ones7 = jnp.ones((1, 7), f32)
    s = _mm(ones7, e)
    logs = jnp.log(s)
    ulogp = -logs
    lp_all = neg - m - logs
    p_all = e / s
    t = jnp.where(ua, p_all * lp_all, 0.0)
    ent = -_mm(ones7, t)
    iota7 = lax.broadcasted_iota(jnp.int32, (7, _P), 0)
    act = jnp.min(jnp.where(neg == m, iota7, 7), axis=0, keepdims=True)

    flogits = _mm(w_fact_ref[...], factf_ref[0]) + b_fact_ref[...]
    fneg = jnp.where(fva, flogits, _NEG)
    fm = jnp.max(fneg, axis=0, keepdims=True)
    iota4 = lax.broadcasted_iota(jnp.int32, (4, _P), 0)
    fact = jnp.min(jnp.where(fneg == fm, iota4, 4), axis=0, keepdims=True)

    sid = jnp.where(umask, loc1_ref[0] + 10, _NIDS)

    ulogp_ref[0] = ulogp
    uent_ref[0] = ent
    critic_ref[0] = critic
    ids_ref[0] = sid
    factmap_ref[0] = jnp.where(fmask, fact.astype(f32), 0.0)
    z = jnp.zeros((1, _P), f32)
    ch0 = jnp.where(umask, act.astype(f32), 0.0)
    ch5 = jnp.where(umask, 1.0, 0.0)
    uactmap_ref[0] = jnp.concatenate([ch0, z, z, z, z, ch5], axis=0)


def _chan_spec(c):
    return pl.BlockSpec((1, c, _P), lambda b: (b, 0, 0))


def _w_spec(shape):
    return pl.BlockSpec(shape, lambda b: (0, 0))


def _dense_call(mapf, unitf, factf, loc1, vfact, vmove, vtrans, vpick, vdig,
                vsd, vrech, vdn, w_au, b_au, w_fact, b_fact):
    out_shapes = (
        jax.ShapeDtypeStruct((_B, 1, _P), jnp.float32),
        jax.ShapeDtypeStruct((_B, 1, _P), jnp.float32),
        jax.ShapeDtypeStruct((_B, 1, _P), jnp.float32),
        jax.ShapeDtypeStruct((_B, 1, _P), jnp.int32),
        jax.ShapeDtypeStruct((_B, 1, _P), jnp.float32),
        jax.ShapeDtypeStruct((_B, 6, _P), jnp.float32),
    )
    in_specs = [
        _chan_spec(2), _chan_spec(3), _chan_spec(6), _chan_spec(1),
        _chan_spec(4), _chan_spec(10), _chan_spec(50), _chan_spec(10),
        _chan_spec(2), _chan_spec(1), _chan_spec(2), _chan_spec(1),
        _w_spec((8, 5)), _w_spec((8, 1)), _w_spec((4, 6)), _w_spec((4, 1)),
    ]
    out_specs = (
        _chan_spec(1), _chan_spec(1), _chan_spec(1), _chan_spec(1),
        _chan_spec(1), _chan_spec(6),
    )
    return pl.pallas_call(
        _dense_body,
        grid=(_B,),
        in_specs=in_specs,
        out_specs=out_specs,
        out_shape=out_shapes,
        compiler_params=pltpu.CompilerParams(
            dimension_semantics=("arbitrary",)),
    )(mapf, unitf, factf, loc1, vfact, vmove, vtrans, vpick, vdig, vsd,
      vrech, vdn, w_au, b_au, w_fact, b_fact)


_NC = 2
_NS = 16
_NW = _NC * _NS
_RPW = _B // _NW
_NGRP = _P // 16
_NZGRP = _NPAD // 16


def _scatter_body(ids_hbm, lp_hbm, cv_hbm, en_hbm,
                  out_lp, out_cv, out_en,
                  ids_v, lp_v, cv_v, en_v, olp_v, ocv_v, oen_v):
    wid = lax.axis_index("s") * _NC + lax.axis_index("c")
    in_base = wid * (_RPW * _P)
    out_base = wid * (_RPW * _NPAD)
    pltpu.sync_copy(ids_hbm.at[pl.ds(in_base, _RPW * _P)], ids_v)
    pltpu.sync_copy(lp_hbm.at[pl.ds(in_base, _RPW * _P)], lp_v)
    pltpu.sync_copy(cv_hbm.at[pl.ds(in_base, _RPW * _P)], cv_v)
    pltpu.sync_copy(en_hbm.at[pl.ds(in_base, _RPW * _P)], en_v)

    zero16 = jnp.zeros((16,), jnp.float32)

    def zgrp(i, _):
        sl = pl.ds(i * 16, 16)
        olp_v[sl] = zero16
        ocv_v[sl] = zero16
        oen_v[sl] = zero16
        return 0
    lax.fori_loop(0, _RPW * _NZGRP, zgrp, 0)

    for r in range(_RPW):
        def sgrp(j, _, r=r):
            sl = pl.ds(r * _P + j * 16, 16)
            idx = ids_v[sl] + (r * _NPAD)
            plsc.store_scatter(olp_v, [idx], lp_v[sl])
            plsc.store_scatter(ocv_v, [idx], cv_v[sl])
            plsc.store_scatter(oen_v, [idx], en_v[sl])
            return 0
        lax.fori_loop(0, _NGRP, sgrp, 0)

    pltpu.sync_copy(olp_v, out_lp.at[pl.ds(out_base, _RPW * _NPAD)])
    pltpu.sync_copy(ocv_v, out_cv.at[pl.ds(out_base, _RPW * _NPAD)])
    pltpu.sync_copy(oen_v, out_en.at[pl.ds(out_base, _RPW * _NPAD)])


def _scatter_call(ids, lp, cv, en):
    fn = pl.kernel(
        _scatter_body,
        mesh=plsc.VectorSubcoreMesh(core_axis_name="c", subcore_axis_name="s"),
        compiler_params=pltpu.CompilerParams(needs_layout_passes=False),
        out_type=[jax.ShapeDtypeStruct((_B * _NPAD,), jnp.float32)] * 3,
        scratch_types=[
            pltpu.VMEM((_RPW * _P,), jnp.int32),
            pltpu.VMEM((_RPW * _P,), jnp.float32),
            pltpu.VMEM((_RPW * _P,), jnp.float32),
            pltpu.VMEM((_RPW * _P,), jnp.float32),
            pltpu.VMEM((_RPW * _NPAD,), jnp.float32),
            pltpu.VMEM((_RPW * _NPAD,), jnp.float32),
            pltpu.VMEM((_RPW * _NPAD,), jnp.float32),
        ],
    )
    return fn(ids, lp, cv, en)


def kernel(global_feature, map_feature, factory_feature, unit_feature,
           location_feature, va_factory_act, va_move, va_transfer, va_pickup,
           va_dig, va_self_destruct, va_recharge, va_do_nothing,
           W_factory, b_factory, W_critic, b_critic, W_dir, b_dir,
           W_act, b_act):
    B, _, H, W = map_feature.shape
    P = H * W

    def chans(x):
        return x.reshape(B, -1, P)

    loc1 = location_feature[:, 1:2].reshape(B, 1, P)
    w_au = jnp.concatenate([W_act, W_critic], axis=0)
    b_au = jnp.concatenate([b_act, b_critic], axis=0).reshape(8, 1)
    ulogp, uent, critic, ids, factmap, uactmap = _dense_call(
        chans(map_feature), chans(unit_feature), chans(factory_feature), loc1,
        chans(va_factory_act), chans(va_move), chans(va_transfer),
        chans(va_pickup), chans(va_dig), chans(va_self_destruct),
        chans(va_recharge), chans(va_do_nothing),
        w_au, b_au, W_factory, b_factory.reshape(4, 1))

    lp_t, cv_t, en_t = _scatter_call(
        ids.reshape(B * P), ulogp.reshape(B * P), critic.reshape(B * P),
        uent.reshape(B * P))

    logp = lp_t.reshape(B, _NPAD)[:, :_NIDS]
    critic_value = cv_t.reshape(B, _NPAD)[:, :_NIDS]
    entropy = en_t.reshape(B, _NPAD)[:, :_NIDS]
    factory_act_map = factmap.reshape(B, H, W)
    unit_act_map = uactmap.reshape(B, 6, H, W)
    return (logp, critic_value, factory_act_map, unit_act_map, entropy)


# speedup vs baseline: 15.4437x; 1.4394x over previous
import functools

import jax
import jax.numpy as jnp
from jax import lax
from jax.experimental import pallas as pl
from jax.experimental.pallas import tpu as pltpu
from jax.experimental.pallas import tpu_sc as plsc

_B = 256
_H = 48
_W = 48
_P = _H * _W
_NIDS = 1000
_NPAD = 1008
_NEG = -1e9


_SL = 18
_LN = 128


def _dense_body(mapf_ref, unitf_ref, factf_ref, loc1_ref,
                vfact_ref, vmove_ref, vtrans_ref, vpick_ref, vdig_ref,
                vsd_ref, vrech_ref, vdn_ref,
                w_au_ref, b_au_ref, w_fact_ref, b_fact_ref,
                ulogp_ref, uent_ref, critic_ref, ids_ref,
                factmap_ref, uactmap_ref):
    f32 = jnp.float32

    def _tree(items, op):
        while len(items) > 1:
            nxt = [op(items[i], items[i + 1])
                   for i in range(0, len(items) - 1, 2)]
            if len(items) % 2:
                nxt.append(items[-1])
            items = nxt
        return items[0]

    def anyi(ref):
        x = ref[0]
        return _tree([x[c] for c in range(x.shape[0])],
                     lambda a, b: a | b)

    ui = [anyi(vmove_ref), anyi(vtrans_ref), anyi(vpick_ref), anyi(vdig_ref),
          vsd_ref[0, 0], anyi(vrech_ref), vdn_ref[0, 0]]
    ua = [u.astype(jnp.int32) != 0 for u in ui]
    umask = _tree(ui, lambda a, b: a | b).astype(jnp.int32) != 0

    val = [mapf_ref[0, 0], mapf_ref[0, 1],
           unitf_ref[0, 0], unitf_ref[0, 1], unitf_ref[0, 2]]

    def head(w_ref, b_ref, feats, k):
        acc = feats[0] * w_ref[k, 0]
        for c in range(1, len(feats)):
            acc = acc + feats[c] * w_ref[k, c]
        return acc + b_ref[k]

    neg = [jnp.where(ua[k], head(w_au_ref, b_au_ref, val, k), _NEG)
           for k in range(7)]
    m = _tree(list(neg), jnp.maximum)
    e = [jnp.exp(neg[k] - m) for k in range(7)]
    s = _tree(list(e), lambda a, b: a + b)
    logs = jnp.log(s)
    inv = 1.0 / s
    mls = m + logs
    ent_terms = [jnp.where(ua[k], (e[k] * inv) * (neg[k] - mls), 0.0)
                 for k in range(7)]
    ent = -_tree(ent_terms, lambda a, b: a + b)
    def amax(a, b):
        gt = b[0] > a[0]
        return jnp.maximum(a[0], b[0]), jnp.where(gt, b[1], a[1])

    _, act = _tree([(neg[k], jnp.int32(k)) for k in range(7)], amax)
    act = jnp.broadcast_to(act, (_SL, _LN))
    critic = head(w_au_ref, b_au_ref, val, 7)

    fva = [vfact_ref[0, k].astype(jnp.int32) != 0 for k in range(4)]
    fmask = ((vfact_ref[0, 0] | vfact_ref[0, 1] | vfact_ref[0, 2]
              | vfact_ref[0, 3]).astype(jnp.int32) != 0)
    ffeat = [factf_ref[0, c] for c in range(6)]
    fneg = [jnp.where(fva[k], head(w_fact_ref, b_fact_ref, ffeat, k), _NEG)
            for k in range(4)]
    _, fact = _tree([(fneg[k], jnp.int32(k)) for k in range(4)], amax)
    fact = jnp.broadcast_to(fact, (_SL, _LN))

    sid = jnp.where(umask, loc1_ref[0, 0] + 10, _NIDS)

    ulogp_ref[0, 0] = -logs
    uent_ref[0, 0] = ent
    critic_ref[0, 0] = critic
    ids_ref[0, 0] = sid
    factmap_ref[0, 0] = jnp.where(fmask, fact.astype(f32), 0.0)
    z = jnp.zeros((_SL, _LN), f32)
    uactmap_ref[0, 0] = jnp.where(umask, act.astype(f32), 0.0)
    uactmap_ref[0, 1] = z
    uactmap_ref[0, 2] = z
    uactmap_ref[0, 3] = z
    uactmap_ref[0, 4] = z
    uactmap_ref[0, 5] = jnp.where(umask, 1.0, 0.0)


def _chan_spec(c):
    return pl.BlockSpec((1, c, _SL, _LN), lambda b: (b, 0, 0, 0))


def _smem_spec(shape):
    return pl.BlockSpec(shape, lambda b: tuple(0 for _ in shape),
                        memory_space=pltpu.SMEM)


def _dense_call(mapf, unitf, factf, loc1, vfact, vmove, vtrans, vpick, vdig,
                vsd, vrech, vdn, w_au, b_au, w_fact, b_fact):
    out_shapes = (
        jax.ShapeDtypeStruct((_B, 1, _SL, _LN), jnp.float32),
        jax.ShapeDtypeStruct((_B, 1, _SL, _LN), jnp.float32),
        jax.ShapeDtypeStruct((_B, 1, _SL, _LN), jnp.float32),
        jax.ShapeDtypeStruct((_B, 1, _SL, _LN), jnp.int32),
        jax.ShapeDtypeStruct((_B, 1, _SL, _LN), jnp.float32),
        jax.ShapeDtypeStruct((_B, 6, _SL, _LN), jnp.float32),
    )
    in_specs = [
        _chan_spec(2), _chan_spec(3), _chan_spec(6), _chan_spec(1),
        _chan_spec(4), _chan_spec(10), _chan_spec(50), _chan_spec(10),
        _chan_spec(2), _chan_spec(1), _chan_spec(2), _chan_spec(1),
        _smem_spec((8, 5)), _smem_spec((8,)), _smem_spec((4, 6)),
        _smem_spec((4,)),
    ]
    out_specs = (
        _chan_spec(1), _chan_spec(1), _chan_spec(1), _chan_spec(1),
        _chan_spec(1), _chan_spec(6),
    )
    return pl.pallas_call(
        _dense_body,
        grid=(_B,),
        in_specs=in_specs,
        out_specs=out_specs,
        out_shape=out_shapes,
        compiler_params=pltpu.CompilerParams(
            dimension_semantics=("arbitrary",)),
    )(mapf, unitf, factf, loc1, vfact, vmove, vtrans, vpick, vdig, vsd,
      vrech, vdn, w_au, b_au, w_fact, b_fact)


_NC = 2
_NS = 16
_NW = _NC * _NS
_RPW = _B // _NW
_NGRP = _P // 16
_NZGRP = _NPAD // 16


def _scatter_body(ids_hbm, lp_hbm, cv_hbm, en_hbm,
                  out_lp, out_cv, out_en,
                  ids_v, lp_v, cv_v, en_v, olp_v, ocv_v, oen_v):
    wid = lax.axis_index("s") * _NC + lax.axis_index("c")
    in_base = wid * (_RPW * _P)
    out_base = wid * (_RPW * _NPAD)
    pltpu.sync_copy(ids_hbm.at[pl.ds(in_base, _RPW * _P)], ids_v)
    pltpu.sync_copy(lp_hbm.at[pl.ds(in_base, _RPW * _P)], lp_v)
    pltpu.sync_copy(cv_hbm.at[pl.ds(in_base, _RPW * _P)], cv_v)
    pltpu.sync_copy(en_hbm.at[pl.ds(in_base, _RPW * _P)], en_v)

    zero16 = jnp.zeros((16,), jnp.float32)

    def zgrp(i, _):
        sl = pl.ds(i * 16, 16)
        olp_v[sl] = zero16
        ocv_v[sl] = zero16
        oen_v[sl] = zero16
        return 0
    lax.fori_loop(0, _RPW * _NZGRP, zgrp, 0)

    for r in range(_RPW):
        def sgrp(j, _, r=r):
            sl = pl.ds(r * _P + j * 16, 16)
            idx = ids_v[sl] + (r * _NPAD)
            plsc.store_scatter(olp_v, [idx], lp_v[sl])
            plsc.store_scatter(ocv_v, [idx], cv_v[sl])
            plsc.store_scatter(oen_v, [idx], en_v[sl])
            return 0
        lax.fori_loop(0, _NGRP, sgrp, 0)

    pltpu.sync_copy(olp_v, out_lp.at[pl.ds(out_base, _RPW * _NPAD)])
    pltpu.sync_copy(ocv_v, out_cv.at[pl.ds(out_base, _RPW * _NPAD)])
    pltpu.sync_copy(oen_v, out_en.at[pl.ds(out_base, _RPW * _NPAD)])


def _scatter_call(ids, lp, cv, en):
    fn = pl.kernel(
        _scatter_body,
        mesh=plsc.VectorSubcoreMesh(core_axis_name="c", subcore_axis_name="s"),
        compiler_params=pltpu.CompilerParams(needs_layout_passes=False),
        out_type=[jax.ShapeDtypeStruct((_B * _NPAD,), jnp.float32)] * 3,
        scratch_types=[
            pltpu.VMEM((_RPW * _P,), jnp.int32),
            pltpu.VMEM((_RPW * _P,), jnp.float32),
            pltpu.VMEM((_RPW * _P,), jnp.float32),
            pltpu.VMEM((_RPW * _P,), jnp.float32),
            pltpu.VMEM((_RPW * _NPAD,), jnp.float32),
            pltpu.VMEM((_RPW * _NPAD,), jnp.float32),
            pltpu.VMEM((_RPW * _NPAD,), jnp.float32),
        ],
    )
    return fn(ids, lp, cv, en)


def kernel(global_feature, map_feature, factory_feature, unit_feature,
           location_feature, va_factory_act, va_move, va_transfer, va_pickup,
           va_dig, va_self_destruct, va_recharge, va_do_nothing,
           W_factory, b_factory, W_critic, b_critic, W_dir, b_dir,
           W_act, b_act):
    B, _, H, W = map_feature.shape
    P = H * W

    def chans(x):
        if x.dtype == jnp.bool_:
            x = x.view(jnp.int8)
        return x.reshape(B, -1, _SL, _LN)

    loc1 = location_feature[:, 1:2].reshape(B, 1, _SL, _LN)
    w_au = jnp.concatenate([W_act, W_critic], axis=0)
    b_au = jnp.concatenate([b_act, b_critic], axis=0)
    ulogp, uent, critic, ids, factmap, uactmap = _dense_call(
        chans(map_feature), chans(unit_feature), chans(factory_feature), loc1,
        chans(va_factory_act), chans(va_move), chans(va_transfer),
        chans(va_pickup), chans(va_dig), chans(va_self_destruct),
        chans(va_recharge), chans(va_do_nothing),
        w_au, b_au, W_factory, b_factory)

    lp_t, cv_t, en_t = _scatter_call(
        ids.reshape(B * P), ulogp.reshape(B * P), critic.reshape(B * P),
        uent.reshape(B * P))

    logp = lp_t.reshape(B, _NPAD)[:, :_NIDS]
    critic_value = cv_t.reshape(B, _NPAD)[:, :_NIDS]
    entropy = en_t.reshape(B, _NPAD)[:, :_NIDS]
    factory_act_map = factmap.reshape(B, H, W)
    unit_act_map = uactmap.reshape(B, 6, H, W)
    return (logp, critic_value, factory_act_map, unit_act_map, entropy)
